# baseline (speedup 1.0000x reference)
import jax
import jax.numpy as jnp
from jax import lax
from jax.experimental import pallas as pl
from jax.experimental.pallas import tpu as pltpu
from jax.experimental.pallas import tpu_sc as plsc

NC, NS, L = 2, 16, 16
NW = NC * NS
B, S, J = 8, 16384, 200
JT, JR = 25, 8
ST, SR = 128, 128
ROWS = B * S
ROWS_PER_W = ROWS // NW
ST_PER_W = ROWS_PER_W // SR
CHUNK_WORDS = ST_PER_W * JR * SR
PAIRS = JR // 2
M2_PAD = 704
INV_LEN = 1.0 / J


def _sc_body(mask2_hbm, seq_hbm, out_hbm, mask2_v, buf0, buf1, acc_v, sem0, sem1):
    wid = lax.axis_index("s") * NC + lax.axis_index("c")
    b = wid // 4
    st0 = (wid % 4) * ST_PER_W
    pltpu.sync_copy(mask2_hbm, mask2_v)
    lane = lax.iota(jnp.int32, L)
    zeros = jnp.zeros((L,), jnp.float32)

    def zero_body(i, tok):
        acc_v[pl.ds(pl.multiple_of(i * L, 8), L)] = zeros
        return tok

    lax.fori_loop(0, ROWS_PER_W // L, zero_body, 0)

    def start(jt, buf, sem):
        src = pl.multiple_of(((b * JT + jt) * ST + st0) * (JR * SR), 8)
        pltpu.async_copy(seq_hbm.at[pl.ds(src, CHUNK_WORDS)], buf, sem)

    def wait(buf, sem):
        pltpu.make_async_copy(
            seq_hbm.at[pl.ds(0, CHUNK_WORDS)], buf, sem
        ).wait()

    def compute(buf):
        def st_body(st, tok):
            sbase = st * (JR * SR)
            abase = st * SR
            for q in range(JR):
                aoff = pl.multiple_of(abase + q * L, 8)
                acc = acc_v[pl.ds(aoff, L)]
                gs = []
                for p in range(PAIRS):
                    offa = pl.multiple_of(sbase + (2 * p) * SR + q * L, 8)
                    offb = pl.multiple_of(sbase + (2 * p + 1) * SR + q * L, 8)
                    a = buf[pl.ds(offa, L)]
                    bb = buf[pl.ds(offb, L)]
                    addr = (a << 9) + (bb << 4) + lane
                    gs.append(plsc.load_gather(mask2_v, [addr]))
                acc_v[pl.ds(aoff, L)] = acc + ((gs[0] + gs[1]) + (gs[2] + gs[3]))
            return tok

        lax.fori_loop(0, ST_PER_W, st_body, 0, unroll=2)

    start(0, buf0, sem0)
    start(1, buf1, sem1)

    def outer(cc, tok):
        jt = cc * 2
        wait(buf0, sem0)
        compute(buf0)

        @pl.when(cc < JT // 2)
        def _():
            start(jt + 2, buf0, sem0)

        wait(buf1, sem1)
        compute(buf1)

        @pl.when(cc < JT // 2 - 1)
        def _():
            start(jt + 3, buf1, sem1)

        return tok

    lax.fori_loop(0, JT // 2, outer, 0)
    wait(buf0, sem0)
    compute(buf0)

    def scale_body(i, tok):
        off = pl.multiple_of(i * L, 8)
        acc_v[pl.ds(off, L)] = acc_v[pl.ds(off, L)] * INV_LEN
        return tok

    lax.fori_loop(0, ROWS_PER_W // L, scale_body, 0)
    pltpu.sync_copy(acc_v, out_hbm.at[pl.ds(wid * ROWS_PER_W, ROWS_PER_W)])


def kernel(seq_samples, reward_mask):
    seq_view = (
        seq_samples.transpose(0, 2, 1)
        .reshape(B, JT, JR, ST, SR)
        .transpose(0, 1, 3, 2, 4)
        .reshape(-1)
    )
    m2 = reward_mask[:, None] + reward_mask[None, :]
    m2 = jnp.pad(m2, ((0, 1), (0, 32 - m2.shape[1])))
    mask2 = jnp.repeat(m2.reshape(-1), L)
    mesh = plsc.VectorSubcoreMesh(
        core_axis_name="c", subcore_axis_name="s", num_cores=NC, num_subcores=NS
    )
    out = pl.kernel(
        _sc_body,
        out_type=jax.ShapeDtypeStruct((ROWS,), jnp.float32),
        mesh=mesh,
        compiler_params=pltpu.CompilerParams(needs_layout_passes=False),
        scratch_types=[
            pltpu.VMEM((M2_PAD * L,), jnp.float32),
            pltpu.VMEM((CHUNK_WORDS,), jnp.int32),
            pltpu.VMEM((CHUNK_WORDS,), jnp.int32),
            pltpu.VMEM((ROWS_PER_W,), jnp.float32),
            pltpu.SemaphoreType.DMA,
            pltpu.SemaphoreType.DMA,
        ],
    )(mask2, seq_view)
    return out.reshape(B, S)


# speedup vs baseline: 1.0085x; 1.0085x over previous
import jax
import jax.numpy as jnp
from jax import lax
from jax.experimental import pallas as pl
from jax.experimental.pallas import tpu as pltpu
from jax.experimental.pallas import tpu_sc as plsc

NC, NS, L = 2, 16, 16
NW = NC * NS
B, S, J = 8, 16384, 200
JT, JR = 25, 8
ST, SR = 128, 128
ROWS = B * S
ROWS_PER_W = ROWS // NW
ST_PER_W = ROWS_PER_W // SR
CHUNK_WORDS = ST_PER_W * JR * SR
PAIRS = JR // 2
M2_PAD = 704
INV_LEN = 1.0 / J


def _sc_body(mask2_hbm, seq_hbm, out_hbm, mask2_v, buf0, buf1, acc_v, sem0, sem1):
    wid = lax.axis_index("s") * NC + lax.axis_index("c")
    b = wid // 4
    st0 = (wid % 4) * ST_PER_W
    pltpu.sync_copy(mask2_hbm, mask2_v)
    lane = lax.iota(jnp.int32, L)
    zeros = jnp.zeros((L,), jnp.float32)

    def zero_body(i, tok):
        acc_v[pl.ds(pl.multiple_of(i * L, 8), L)] = zeros
        return tok

    lax.fori_loop(0, ROWS_PER_W // L, zero_body, 0)

    def start(jt, buf, sem):
        src = pl.multiple_of(((b * JT + jt) * ST + st0) * (JR * SR), 8)
        pltpu.async_copy(seq_hbm.at[pl.ds(src, CHUNK_WORDS)], buf, sem)

    def wait(buf, sem):
        pltpu.make_async_copy(
            seq_hbm.at[pl.ds(0, CHUNK_WORDS)], buf, sem
        ).wait()

    def compute(buf):
        def st_body(st, tok):
            sbase = st * (JR * SR)
            abase = st * SR
            for q in range(JR):
                aoff = pl.multiple_of(abase + q * L, 8)
                acc = acc_v[pl.ds(aoff, L)]
                gs = []
                for p in range(PAIRS):
                    offa = pl.multiple_of(sbase + (2 * p) * SR + q * L, 8)
                    offb = pl.multiple_of(sbase + (2 * p + 1) * SR + q * L, 8)
                    a = buf[pl.ds(offa, L)]
                    bb = buf[pl.ds(offb, L)]
                    addr = (a << 9) + (bb << 4) + lane
                    gs.append(plsc.load_gather(mask2_v, [addr]))
                acc_v[pl.ds(aoff, L)] = acc + ((gs[0] + gs[1]) + (gs[2] + gs[3]))
            return tok

        lax.fori_loop(0, ST_PER_W, st_body, 0)

    start(0, buf0, sem0)
    start(1, buf1, sem1)

    def outer(cc, tok):
        jt = cc * 2
        wait(buf0, sem0)
        compute(buf0)

        @pl.when(cc < JT // 2)
        def _():
            start(jt + 2, buf0, sem0)

        wait(buf1, sem1)
        compute(buf1)

        @pl.when(cc < JT // 2 - 1)
        def _():
            start(jt + 3, buf1, sem1)

        return tok

    lax.fori_loop(0, JT // 2, outer, 0)
    wait(buf0, sem0)
    compute(buf0)

    def scale_body(i, tok):
        off = pl.multiple_of(i * L, 8)
        acc_v[pl.ds(off, L)] = acc_v[pl.ds(off, L)] * INV_LEN
        return tok

    lax.fori_loop(0, ROWS_PER_W // L, scale_body, 0)
    pltpu.sync_copy(acc_v, out_hbm.at[pl.ds(wid * ROWS_PER_W, ROWS_PER_W)])


def kernel(seq_samples, reward_mask):
    seq_view = (
        seq_samples.transpose(0, 2, 1)
        .reshape(B, JT, JR, ST, SR)
        .transpose(0, 1, 3, 2, 4)
        .reshape(-1)
    )
    m2 = reward_mask[:, None] + reward_mask[None, :]
    m2 = jnp.pad(m2, ((0, 1), (0, 32 - m2.shape[1])))
    mask2 = jnp.repeat(m2.reshape(-1), L)
    mesh = plsc.VectorSubcoreMesh(
        core_axis_name="c", subcore_axis_name="s", num_cores=NC, num_subcores=NS
    )
    out = pl.kernel(
        _sc_body,
        out_type=jax.ShapeDtypeStruct((ROWS,), jnp.float32),
        mesh=mesh,
        compiler_params=pltpu.CompilerParams(needs_layout_passes=False),
        scratch_types=[
            pltpu.VMEM((M2_PAD * L,), jnp.float32),
            pltpu.VMEM((CHUNK_WORDS,), jnp.int32),
            pltpu.VMEM((CHUNK_WORDS,), jnp.int32),
            pltpu.VMEM((ROWS_PER_W,), jnp.float32),
            pltpu.SemaphoreType.DMA,
            pltpu.SemaphoreType.DMA,
        ],
    )(mask2, seq_view)
    return out.reshape(B, S)


# speedup vs baseline: 1.1624x; 1.1526x over previous
import jax
import jax.numpy as jnp
from jax import lax
from jax.experimental import pallas as pl
from jax.experimental.pallas import tpu as pltpu
from jax.experimental.pallas import tpu_sc as plsc

NC, NS, L = 2, 16, 16
NW = NC * NS
B, S, J = 8, 16384, 200
JT, JR = 25, 8
ST, SR = 128, 128
ROWS = B * S
ROWS_PER_W = ROWS // NW
ST_PER_W = ROWS_PER_W // SR
CHUNK_WORDS = ST_PER_W * JR * SR
PAIRS = JR // 2
M2_PAD = 704
INV_LEN = 1.0 / J


def _sc_body(mask2_hbm, seq_hbm, out_hbm, mask2_v, buf0, buf1, acc_v, sem0, sem1):
    wid = lax.axis_index("s") * NC + lax.axis_index("c")
    b = wid // 4
    st0 = (wid % 4) * ST_PER_W
    pltpu.sync_copy(mask2_hbm, mask2_v)
    lane = lax.iota(jnp.int32, L)
    zeros = jnp.zeros((L,), jnp.float32)

    def zero_body(i, tok):
        acc_v[pl.ds(pl.multiple_of(i * L, 8), L)] = zeros
        return tok

    lax.fori_loop(0, ROWS_PER_W // L, zero_body, 0)

    def start(jt, buf, sem):
        src = pl.multiple_of(((b * JT + jt) * ST + st0) * (JR * SR), 8)
        pltpu.async_copy(seq_hbm.at[pl.ds(src, CHUNK_WORDS)], buf, sem)

    def wait(buf, sem):
        pltpu.make_async_copy(
            seq_hbm.at[pl.ds(0, CHUNK_WORDS)], buf, sem
        ).wait()

    def compute(buf):
        def st_body(st, tok):
            sbase = st * (JR * SR)
            abase = st * SR
            for q in range(JR):
                aoff = pl.multiple_of(abase + q * L, 8)
                acc = acc_v[pl.ds(aoff, L)]
                for p in range(PAIRS):
                    offa = pl.multiple_of(sbase + (2 * p) * SR + q * L, 8)
                    offb = pl.multiple_of(sbase + (2 * p + 1) * SR + q * L, 8)
                    a = buf[pl.ds(offa, L)]
                    bb = buf[pl.ds(offb, L)]
                    addr = ((a << 5) + bb) * L + lane
                    acc = acc + plsc.load_gather(mask2_v, [addr])
                acc_v[pl.ds(aoff, L)] = acc
            return tok

        lax.fori_loop(0, ST_PER_W, st_body, 0)

    start(0, buf0, sem0)
    start(1, buf1, sem1)

    def outer(cc, tok):
        jt = cc * 2
        wait(buf0, sem0)
        compute(buf0)

        @pl.when(cc < JT // 2)
        def _():
            start(jt + 2, buf0, sem0)

        wait(buf1, sem1)
        compute(buf1)

        @pl.when(cc < JT // 2 - 1)
        def _():
            start(jt + 3, buf1, sem1)

        return tok

    lax.fori_loop(0, JT // 2, outer, 0)
    wait(buf0, sem0)
    compute(buf0)

    def scale_body(i, tok):
        off = pl.multiple_of(i * L, 8)
        acc_v[pl.ds(off, L)] = acc_v[pl.ds(off, L)] * INV_LEN
        return tok

    lax.fori_loop(0, ROWS_PER_W // L, scale_body, 0)
    pltpu.sync_copy(acc_v, out_hbm.at[pl.ds(wid * ROWS_PER_W, ROWS_PER_W)])


def kernel(seq_samples, reward_mask):
    seq_view = (
        seq_samples.transpose(0, 2, 1)
        .reshape(B, JT, JR, ST, SR)
        .transpose(0, 1, 3, 2, 4)
        .reshape(-1)
    )
    m2 = reward_mask[:, None] + reward_mask[None, :]
    m2 = jnp.pad(m2, ((0, 1), (0, 32 - m2.shape[1])))
    mask2 = jnp.repeat(m2.reshape(-1), L)
    mesh = plsc.VectorSubcoreMesh(
        core_axis_name="c", subcore_axis_name="s", num_cores=NC, num_subcores=NS
    )
    out = pl.kernel(
        _sc_body,
        out_type=jax.ShapeDtypeStruct((ROWS,), jnp.float32),
        mesh=mesh,
        compiler_params=pltpu.CompilerParams(needs_layout_passes=False),
        scratch_types=[
            pltpu.VMEM((M2_PAD * L,), jnp.float32),
            pltpu.VMEM((CHUNK_WORDS,), jnp.int32),
            pltpu.VMEM((CHUNK_WORDS,), jnp.int32),
            pltpu.VMEM((ROWS_PER_W,), jnp.float32),
            pltpu.SemaphoreType.DMA,
            pltpu.SemaphoreType.DMA,
        ],
    )(mask2, seq_view)
    return out.reshape(B, S)


# speedup vs baseline: 1.4455x; 1.2436x over previous
import jax
import jax.numpy as jnp
from jax import lax
from jax.experimental import pallas as pl
from jax.experimental.pallas import tpu as pltpu
from jax.experimental.pallas import tpu_sc as plsc

NC, NS, L = 2, 16, 16
NW = NC * NS
B, S, J = 8, 16384, 200
JT, JR = 25, 8
ST, SR = 128, 128
PAIRS = JR // 2
M2_PAD = 704
INV_LEN = 1.0 / J

ST_SC = 32
ST_PER_W = ST_SC // 4
ROWS_PER_W = ST_PER_W * SR
SC_ROWS = B * ST_SC * SR
CHUNK_WORDS = ST_PER_W * JR * SR

TC_LANES = 1024
TC_S0 = ST_SC * SR
TC_NSB = (S - TC_S0) // TC_LANES


def _sc_body(mask2_hbm, seq_hbm, out_hbm, mask2_v, buf0, buf1, acc_v, sem0, sem1):
    wid = lax.axis_index("s") * NC + lax.axis_index("c")
    b = wid // 4
    st0 = (wid % 4) * ST_PER_W
    pltpu.sync_copy(mask2_hbm, mask2_v)
    lane = lax.iota(jnp.int32, L)
    zeros = jnp.zeros((L,), jnp.float32)

    def zero_body(i, tok):
        acc_v[pl.ds(pl.multiple_of(i * L, 8), L)] = zeros
        return tok

    lax.fori_loop(0, ROWS_PER_W // L, zero_body, 0)

    def start(jt, buf, sem):
        src = pl.multiple_of(((b * JT + jt) * ST + st0) * (JR * SR), 8)
        pltpu.async_copy(seq_hbm.at[pl.ds(src, CHUNK_WORDS)], buf, sem)

    def wait(buf, sem):
        pltpu.make_async_copy(
            seq_hbm.at[pl.ds(0, CHUNK_WORDS)], buf, sem
        ).wait()

    def compute(buf):
        def st_body(st, tok):
            sbase = st * (JR * SR)
            abase = st * SR
            for q in range(JR):
                aoff = pl.multiple_of(abase + q * L, 8)
                acc = acc_v[pl.ds(aoff, L)]
                for p in range(PAIRS):
                    offa = pl.multiple_of(sbase + (2 * p) * SR + q * L, 8)
                    offb = pl.multiple_of(sbase + (2 * p + 1) * SR + q * L, 8)
                    a = buf[pl.ds(offa, L)]
                    bb = buf[pl.ds(offb, L)]
                    addr = ((a << 5) + bb) * L + lane
                    acc = acc + plsc.load_gather(mask2_v, [addr])
                acc_v[pl.ds(aoff, L)] = acc
            return tok

        lax.fori_loop(0, ST_PER_W, st_body, 0)

    start(0, buf0, sem0)
    start(1, buf1, sem1)

    def outer(cc, tok):
        jt = cc * 2
        wait(buf0, sem0)
        compute(buf0)

        @pl.when(cc < JT // 2)
        def _():
            start(jt + 2, buf0, sem0)

        wait(buf1, sem1)
        compute(buf1)

        @pl.when(cc < JT // 2 - 1)
        def _():
            start(jt + 3, buf1, sem1)

        return tok

    lax.fori_loop(0, JT // 2, outer, 0)
    wait(buf0, sem0)
    compute(buf0)

    def scale_body(i, tok):
        off = pl.multiple_of(i * L, 8)
        acc_v[pl.ds(off, L)] = acc_v[pl.ds(off, L)] * INV_LEN
        return tok

    lax.fori_loop(0, ROWS_PER_W // L, scale_body, 0)
    pltpu.sync_copy(acc_v, out_hbm.at[pl.ds(wid * ROWS_PER_W, ROWS_PER_W)])


def _tc_body(mask_ref, x_ref, o_ref):
    tbl = jnp.broadcast_to(mask_ref[...].reshape(1, 21), (J, 21))
    hits = jnp.take_along_axis(
        tbl, x_ref[0], axis=1, mode=lax.GatherScatterMode.PROMISE_IN_BOUNDS
    )
    o_ref[...] = (jnp.sum(hits, axis=0) * INV_LEN).reshape(1, 1, TC_LANES)


def kernel(seq_samples, reward_mask):
    seq_t = seq_samples.transpose(0, 2, 1)
    seq_view = (
        seq_t.reshape(B, JT, JR, ST, SR)
        .transpose(0, 1, 3, 2, 4)
        .reshape(-1)
    )
    m2 = reward_mask[:, None] + reward_mask[None, :]
    m2 = jnp.pad(m2, ((0, 1), (0, 32 - m2.shape[1])))
    mask2 = jnp.repeat(m2.reshape(-1), L)

    mesh = plsc.VectorSubcoreMesh(
        core_axis_name="c", subcore_axis_name="s", num_cores=NC, num_subcores=NS
    )
    sc_out = pl.kernel(
        _sc_body,
        out_type=jax.ShapeDtypeStruct((SC_ROWS,), jnp.float32),
        mesh=mesh,
        compiler_params=pltpu.CompilerParams(needs_layout_passes=False),
        scratch_types=[
            pltpu.VMEM((M2_PAD * L,), jnp.float32),
            pltpu.VMEM((CHUNK_WORDS,), jnp.int32),
            pltpu.VMEM((CHUNK_WORDS,), jnp.int32),
            pltpu.VMEM((ROWS_PER_W,), jnp.float32),
            pltpu.SemaphoreType.DMA,
            pltpu.SemaphoreType.DMA,
        ],
    )(mask2, seq_view)

    tc_out = pl.pallas_call(
        _tc_body,
        out_shape=jax.ShapeDtypeStruct((B * TC_NSB, 1, TC_LANES), jnp.float32),
        grid=(B, TC_NSB),
        in_specs=[
            pl.BlockSpec((1, 21), lambda b, sb: (0, 0)),
            pl.BlockSpec(
                (1, J, TC_LANES),
                lambda b, sb: (b, 0, sb + TC_S0 // TC_LANES),
            ),
        ],
        out_specs=pl.BlockSpec(
            (1, 1, TC_LANES), lambda b, sb: (b * TC_NSB + sb, 0, 0)
        ),
    )(reward_mask.reshape(1, 21), seq_t)

    return jnp.concatenate(
        [sc_out.reshape(B, ST_SC * SR), tc_out.reshape(B, S - TC_S0)], axis=1
    )


# speedup vs baseline: 1.4875x; 1.0290x over previous
import jax
import jax.numpy as jnp
from jax import lax
from jax.experimental import pallas as pl
from jax.experimental.pallas import tpu as pltpu
from jax.experimental.pallas import tpu_sc as plsc

NC, NS, L = 2, 16, 16
NW = NC * NS
B, S, J = 8, 16384, 200
JT, JR = 25, 8
ST, SR = 128, 128
PAIRS = JR // 2
M2_PAD = 704
INV_LEN = 1.0 / J

ST_SC = 68
ST_PER_W = ST_SC // 4
ROWS_PER_W = ST_PER_W * SR
SC_ROWS = B * ST_SC * SR
CHUNK_WORDS = ST_PER_W * JR * SR

TC_LANES = 512
TC_S0 = ST_SC * SR
TC_NSB = (S - TC_S0) // TC_LANES


def _sc_body(mask2_hbm, seq_hbm, out_hbm, mask2_v, buf0, buf1, acc_v, sem0, sem1):
    wid = lax.axis_index("s") * NC + lax.axis_index("c")
    b = wid // 4
    st0 = (wid % 4) * ST_PER_W
    pltpu.sync_copy(mask2_hbm, mask2_v)
    lane = lax.iota(jnp.int32, L)
    zeros = jnp.zeros((L,), jnp.float32)

    def zero_body(i, tok):
        acc_v[pl.ds(pl.multiple_of(i * L, 8), L)] = zeros
        return tok

    lax.fori_loop(0, ROWS_PER_W // L, zero_body, 0)

    def start(jt, buf, sem):
        src = pl.multiple_of(((b * JT + jt) * ST + st0) * (JR * SR), 8)
        pltpu.async_copy(seq_hbm.at[pl.ds(src, CHUNK_WORDS)], buf, sem)

    def wait(buf, sem):
        pltpu.make_async_copy(
            seq_hbm.at[pl.ds(0, CHUNK_WORDS)], buf, sem
        ).wait()

    def compute(buf):
        def st_body(st, tok):
            sbase = st * (JR * SR)
            abase = st * SR
            for q in range(JR):
                aoff = pl.multiple_of(abase + q * L, 8)
                acc = acc_v[pl.ds(aoff, L)]
                for p in range(PAIRS):
                    offa = pl.multiple_of(sbase + (2 * p) * SR + q * L, 8)
                    offb = pl.multiple_of(sbase + (2 * p + 1) * SR + q * L, 8)
                    a = buf[pl.ds(offa, L)]
                    bb = buf[pl.ds(offb, L)]
                    addr = ((a << 5) + bb) * L + lane
                    acc = acc + plsc.load_gather(mask2_v, [addr])
                acc_v[pl.ds(aoff, L)] = acc
            return tok

        lax.fori_loop(0, ST_PER_W, st_body, 0)

    start(0, buf0, sem0)
    start(1, buf1, sem1)

    def outer(cc, tok):
        jt = cc * 2
        wait(buf0, sem0)
        compute(buf0)

        @pl.when(cc < JT // 2)
        def _():
            start(jt + 2, buf0, sem0)

        wait(buf1, sem1)
        compute(buf1)

        @pl.when(cc < JT // 2 - 1)
        def _():
            start(jt + 3, buf1, sem1)

        return tok

    lax.fori_loop(0, JT // 2, outer, 0)
    wait(buf0, sem0)
    compute(buf0)

    def scale_body(i, tok):
        off = pl.multiple_of(i * L, 8)
        acc_v[pl.ds(off, L)] = acc_v[pl.ds(off, L)] * INV_LEN
        return tok

    lax.fori_loop(0, ROWS_PER_W // L, scale_body, 0)
    pltpu.sync_copy(acc_v, out_hbm.at[pl.ds(wid * ROWS_PER_W, ROWS_PER_W)])


def _tc_body(mask_ref, x_ref, o_ref):
    tbl = jnp.broadcast_to(mask_ref[...].reshape(1, 21), (J, 21))
    hits = jnp.take_along_axis(
        tbl, x_ref[0], axis=1, mode=lax.GatherScatterMode.PROMISE_IN_BOUNDS
    )
    o_ref[...] = (jnp.sum(hits, axis=0) * INV_LEN).reshape(1, 1, TC_LANES)


def kernel(seq_samples, reward_mask):
    seq_t = seq_samples.transpose(0, 2, 1)
    seq_view = (
        seq_t.reshape(B, JT, JR, ST, SR)
        .transpose(0, 1, 3, 2, 4)
        .reshape(-1)
    )
    m2 = reward_mask[:, None] + reward_mask[None, :]
    m2 = jnp.pad(m2, ((0, 1), (0, 32 - m2.shape[1])))
    mask2 = jnp.repeat(m2.reshape(-1), L)

    mesh = plsc.VectorSubcoreMesh(
        core_axis_name="c", subcore_axis_name="s", num_cores=NC, num_subcores=NS
    )
    sc_out = pl.kernel(
        _sc_body,
        out_type=jax.ShapeDtypeStruct((SC_ROWS,), jnp.float32),
        mesh=mesh,
        compiler_params=pltpu.CompilerParams(needs_layout_passes=False),
        scratch_types=[
            pltpu.VMEM((M2_PAD * L,), jnp.float32),
            pltpu.VMEM((CHUNK_WORDS,), jnp.int32),
            pltpu.VMEM((CHUNK_WORDS,), jnp.int32),
            pltpu.VMEM((ROWS_PER_W,), jnp.float32),
            pltpu.SemaphoreType.DMA,
            pltpu.SemaphoreType.DMA,
        ],
    )(mask2, seq_view)

    tc_out = pl.pallas_call(
        _tc_body,
        out_shape=jax.ShapeDtypeStruct((B * TC_NSB, 1, TC_LANES), jnp.float32),
        grid=(B, TC_NSB),
        in_specs=[
            pl.BlockSpec((1, 21), lambda b, sb: (0, 0)),
            pl.BlockSpec(
                (1, J, TC_LANES),
                lambda b, sb: (b, 0, sb + TC_S0 // TC_LANES),
            ),
        ],
        out_specs=pl.BlockSpec(
            (1, 1, TC_LANES), lambda b, sb: (b * TC_NSB + sb, 0, 0)
        ),
    )(reward_mask.reshape(1, 21), seq_t)

    return jnp.concatenate(
        [sc_out.reshape(B, ST_SC * SR), tc_out.reshape(B, S - TC_S0)], axis=1
    )
